# baseline (speedup 1.0000x reference)
import jax
import jax.numpy as jnp
from jax.experimental import pallas as pl

N_TOP = 10
N_BOTTOM = 10
ROWS_PER_BLOCK = 8
N_COLS = 32768
N_SLABS = 16
SLAB_W = N_COLS // N_SLABS


def _oems_pairs(n):
    pairs = []

    def merge(lo, m, r):
        step = r * 2
        if step < m:
            merge(lo, m, step)
            merge(lo + r, m, step)
            for i in range(lo + r, lo + m - r, step):
                pairs.append((i, i + r))
        else:
            pairs.append((lo, lo + r))

    def sort(lo, m):
        if m > 1:
            h = m // 2
            sort(lo, h)
            sort(lo + h, h)
            merge(lo, m, 1)

    sort(0, n)
    return pairs


_SORT16 = _oems_pairs(N_SLABS)


def _bitonic_cleanup(w, asc):
    for d in (8, 4, 2, 1):
        for i in range(16):
            if (i & d) == 0 and i + d < 16:
                a, b = w[i], w[i + d]
                if b is None:
                    continue
                if a is None:
                    w[i], w[i + d] = b, None
                    continue
                if asc:
                    w[i], w[i + d] = jnp.minimum(a, b), jnp.maximum(a, b)
                else:
                    w[i], w[i + d] = jnp.maximum(a, b), jnp.minimum(a, b)
    return w


def _merge_top(a, b):
    k = N_TOP
    m = [jnp.maximum(a[i], b[k - 1 - i]) for i in range(k)]
    w = _bitonic_cleanup(m + [None] * (16 - k), asc=True)
    return [w[k - 1 - i] for i in range(k)]


def _merge_bot(a, b):
    k = N_BOTTOM
    m = [jnp.minimum(a[i], b[k - 1 - i]) for i in range(k)]
    w = _bitonic_cleanup(m + [None] * (16 - k), asc=False)
    return [w[k - 1 - i] for i in range(k)]


def _extreme_block(x_ref, o_ref):
    slabs = [x_ref[:, t * SLAB_W:(t + 1) * SLAB_W] for t in range(N_SLABS)]
    for i, j in _SORT16:
        a, b = slabs[i], slabs[j]
        slabs[i] = jnp.minimum(a, b)
        slabs[j] = jnp.maximum(a, b)
    tops = [slabs[N_SLABS - 1 - i] for i in range(N_TOP)]
    bots = slabs[:N_BOTTOM]

    width = SLAB_W
    while width > 1:
        half = width // 2
        tops = _merge_top([t[:, :half] for t in tops],
                          [t[:, half:width] for t in tops])
        bots = _merge_bot([b[:, :half] for b in bots],
                          [b[:, half:width] for b in bots])
        width = half

    o_ref[...] = jnp.concatenate(tops + bots, axis=1)


@jax.jit
def kernel(x):
    n_rows = x.shape[0]
    grid = (n_rows // ROWS_PER_BLOCK,)
    return pl.pallas_call(
        _extreme_block,
        grid=grid,
        in_specs=[pl.BlockSpec((ROWS_PER_BLOCK, N_COLS), lambda i: (i, 0))],
        out_specs=pl.BlockSpec((ROWS_PER_BLOCK, N_TOP + N_BOTTOM), lambda i: (i, 0)),
        out_shape=jax.ShapeDtypeStruct((n_rows, N_TOP + N_BOTTOM), x.dtype),
    )(x)


# speedup vs baseline: 1.0653x; 1.0653x over previous
import jax
import jax.numpy as jnp
from jax.experimental import pallas as pl
from jax.experimental.pallas import tpu as pltpu

N_TOP = 10
N_BOTTOM = 10
ROWS_PER_BLOCK = 8
N_COLS = 32768
GROUP = 16
LANES = 128
N_GROUPS = N_COLS // (GROUP * LANES)
LIST_W = N_GROUPS * LANES


def _oems_pairs(n):
    pairs = []

    def merge(lo, m, r):
        step = r * 2
        if step < m:
            merge(lo, m, step)
            merge(lo + r, m, step)
            for i in range(lo + r, lo + m - r, step):
                pairs.append((i, i + r))
        else:
            pairs.append((lo, lo + r))

    def sort(lo, m):
        if m > 1:
            h = m // 2
            sort(lo, h)
            sort(lo + h, h)
            merge(lo, m, 1)

    sort(0, n)
    return pairs


_SORT16 = _oems_pairs(GROUP)


def _bitonic_cleanup(w, asc):
    for d in (8, 4, 2, 1):
        for i in range(16):
            if (i & d) == 0 and i + d < 16:
                a, b = w[i], w[i + d]
                if b is None:
                    continue
                if a is None:
                    w[i], w[i + d] = b, None
                    continue
                if asc:
                    w[i], w[i + d] = jnp.minimum(a, b), jnp.maximum(a, b)
                else:
                    w[i], w[i + d] = jnp.maximum(a, b), jnp.minimum(a, b)
    return w


def _merge_top(a, b):
    k = N_TOP
    m = [jnp.maximum(a[i], b[k - 1 - i]) for i in range(k)]
    w = _bitonic_cleanup(m + [None] * (16 - k), asc=True)
    return [w[k - 1 - i] for i in range(k)]


def _merge_bot(a, b):
    k = N_BOTTOM
    m = [jnp.minimum(a[i], b[k - 1 - i]) for i in range(k)]
    w = _bitonic_cleanup(m + [None] * (16 - k), asc=False)
    return [w[k - 1 - i] for i in range(k)]


def _extreme_block(x_ref, o_ref, t_scr, b_scr):
    for g in range(N_GROUPS):
        base = g * GROUP * LANES
        tiles = [x_ref[:, base + t * LANES:base + (t + 1) * LANES]
                 for t in range(GROUP)]
        for i, j in _SORT16:
            a, b = tiles[i], tiles[j]
            tiles[i] = jnp.minimum(a, b)
            tiles[j] = jnp.maximum(a, b)
        sl = slice(g * LANES, (g + 1) * LANES)
        for i in range(N_TOP):
            t_scr[i, :, sl] = tiles[GROUP - 1 - i]
        for i in range(N_BOTTOM):
            b_scr[i, :, sl] = tiles[i]

    tops = [t_scr[i] for i in range(N_TOP)]
    bots = [b_scr[i] for i in range(N_BOTTOM)]
    width = LIST_W
    while width > 1:
        half = width // 2
        tops = _merge_top([t[:, :half] for t in tops],
                          [t[:, half:width] for t in tops])
        bots = _merge_bot([b[:, :half] for b in bots],
                          [b[:, half:width] for b in bots])
        width = half

    o_ref[...] = jnp.concatenate(tops + bots, axis=1)


@jax.jit
def kernel(x):
    n_rows = x.shape[0]
    grid = (n_rows // ROWS_PER_BLOCK,)
    return pl.pallas_call(
        _extreme_block,
        grid=grid,
        in_specs=[pl.BlockSpec((ROWS_PER_BLOCK, N_COLS), lambda i: (i, 0))],
        out_specs=pl.BlockSpec((ROWS_PER_BLOCK, N_TOP + N_BOTTOM), lambda i: (i, 0)),
        out_shape=jax.ShapeDtypeStruct((n_rows, N_TOP + N_BOTTOM), x.dtype),
        scratch_shapes=[
            pltpu.VMEM((N_TOP, ROWS_PER_BLOCK, LIST_W), jnp.float32),
            pltpu.VMEM((N_BOTTOM, ROWS_PER_BLOCK, LIST_W), jnp.float32),
        ],
    )(x)


# speedup vs baseline: 1.1761x; 1.1040x over previous
import jax
import jax.numpy as jnp
from jax.experimental import pallas as pl
from jax.experimental.pallas import tpu as pltpu

N_TOP = 10
N_BOTTOM = 10
ROWS_PER_BLOCK = 8
N_COLS = 32768
GROUP = 16
LANES = 128
N_GROUPS = N_COLS // (GROUP * LANES)
LIST_W = N_GROUPS * LANES


def _oems_pairs(n):
    pairs = []

    def merge(lo, m, r):
        step = r * 2
        if step < m:
            merge(lo, m, step)
            merge(lo + r, m, step)
            for i in range(lo + r, lo + m - r, step):
                pairs.append((i, i + r))
        else:
            pairs.append((lo, lo + r))

    def sort(lo, m):
        if m > 1:
            h = m // 2
            sort(lo, h)
            sort(lo + h, h)
            merge(lo, m, 1)

    sort(0, n)
    return pairs


_SORT16 = _oems_pairs(GROUP)


def _bitonic_cleanup(w, asc):
    for d in (8, 4, 2, 1):
        for i in range(16):
            if (i & d) == 0 and i + d < 16:
                a, b = w[i], w[i + d]
                if b is None:
                    continue
                if a is None:
                    w[i], w[i + d] = b, None
                    continue
                if asc:
                    w[i], w[i + d] = jnp.minimum(a, b), jnp.maximum(a, b)
                else:
                    w[i], w[i + d] = jnp.maximum(a, b), jnp.minimum(a, b)
    return w


def _merge_top(a, b):
    k = N_TOP
    m = [jnp.maximum(a[i], b[k - 1 - i]) for i in range(k)]
    w = _bitonic_cleanup(m + [None] * (16 - k), asc=True)
    return [w[k - 1 - i] for i in range(k)]


def _merge_bot(a, b):
    k = N_BOTTOM
    m = [jnp.minimum(a[i], b[k - 1 - i]) for i in range(k)]
    w = _bitonic_cleanup(m + [None] * (16 - k), asc=False)
    return [w[k - 1 - i] for i in range(k)]


def _extreme_block(x_ref, o_ref, t_scr, b_scr, acc_t, acc_b):
    for g in range(N_GROUPS):
        base = g * GROUP * LANES
        tiles = [x_ref[:, base + t * LANES:base + (t + 1) * LANES]
                 for t in range(GROUP)]
        for i, j in _SORT16:
            a, b = tiles[i], tiles[j]
            tiles[i] = jnp.minimum(a, b)
            tiles[j] = jnp.maximum(a, b)
        sl = slice(g * LANES, (g + 1) * LANES)
        for i in range(N_TOP):
            t_scr[i, :, sl] = tiles[GROUP - 1 - i]
        for i in range(N_BOTTOM):
            b_scr[i, :, sl] = tiles[i]

    tops = [t_scr[i] for i in range(N_TOP)]
    bots = [b_scr[i] for i in range(N_BOTTOM)]
    width = LIST_W
    while width > LANES:
        half = width // 2
        tops = _merge_top([t[:, :half] for t in tops],
                          [t[:, half:width] for t in tops])
        bots = _merge_bot([b[:, :half] for b in bots],
                          [b[:, half:width] for b in bots])
        width = half

    blk = pl.program_id(0)
    row0 = blk * ROWS_PER_BLOCK
    for i in range(N_TOP):
        acc_t[i, pl.ds(row0, ROWS_PER_BLOCK), :] = tops[i]
    for i in range(N_BOTTOM):
        acc_b[i, pl.ds(row0, ROWS_PER_BLOCK), :] = bots[i]

    @pl.when(blk == pl.num_programs(0) - 1)
    def _tail():
        tops = [acc_t[i] for i in range(N_TOP)]
        bots = [acc_b[i] for i in range(N_BOTTOM)]
        width = LANES
        while width > 1:
            half = width // 2
            tops = _merge_top([t[:, :half] for t in tops],
                              [t[:, half:width] for t in tops])
            bots = _merge_bot([b[:, :half] for b in bots],
                              [b[:, half:width] for b in bots])
            width = half
        o_ref[...] = jnp.concatenate(tops + bots, axis=1)


@jax.jit
def kernel(x):
    n_rows = x.shape[0]
    grid = (n_rows // ROWS_PER_BLOCK,)
    return pl.pallas_call(
        _extreme_block,
        grid=grid,
        in_specs=[pl.BlockSpec((ROWS_PER_BLOCK, N_COLS), lambda i: (i, 0))],
        out_specs=pl.BlockSpec((n_rows, N_TOP + N_BOTTOM), lambda i: (0, 0)),
        out_shape=jax.ShapeDtypeStruct((n_rows, N_TOP + N_BOTTOM), x.dtype),
        scratch_shapes=[
            pltpu.VMEM((N_TOP, ROWS_PER_BLOCK, LIST_W), jnp.float32),
            pltpu.VMEM((N_BOTTOM, ROWS_PER_BLOCK, LIST_W), jnp.float32),
            pltpu.VMEM((N_TOP, 128, LANES), jnp.float32),
            pltpu.VMEM((N_BOTTOM, 128, LANES), jnp.float32),
        ],
    )(x)


# speedup vs baseline: 1.4108x; 1.1996x over previous
import jax
import jax.numpy as jnp
from jax.experimental import pallas as pl
from jax.experimental.pallas import tpu as pltpu

N_TOP = 10
N_BOTTOM = 10
ROWS_PER_BLOCK = 16
N_COLS = 32768
GROUP = 16
LANES = 128
N_GROUPS = N_COLS // (GROUP * LANES)
LIST_W = N_GROUPS * LANES


def _oems_pairs(n):
    pairs = []

    def merge(lo, m, r):
        step = r * 2
        if step < m:
            merge(lo, m, step)
            merge(lo + r, m, step)
            for i in range(lo + r, lo + m - r, step):
                pairs.append((i, i + r))
        else:
            pairs.append((lo, lo + r))

    def sort(lo, m):
        if m > 1:
            h = m // 2
            sort(lo, h)
            sort(lo + h, h)
            merge(lo, m, 1)

    sort(0, n)
    return pairs


_SORT16 = _oems_pairs(GROUP)


def _bitonic_cleanup(w, asc):
    for d in (8, 4, 2, 1):
        for i in range(16):
            if (i & d) == 0 and i + d < 16:
                a, b = w[i], w[i + d]
                if b is None:
                    continue
                if a is None:
                    w[i], w[i + d] = b, None
                    continue
                if asc:
                    w[i], w[i + d] = jnp.minimum(a, b), jnp.maximum(a, b)
                else:
                    w[i], w[i + d] = jnp.maximum(a, b), jnp.minimum(a, b)
    return w


def _merge_top(a, b):
    k = N_TOP
    m = [jnp.maximum(a[i], b[k - 1 - i]) for i in range(k)]
    w = _bitonic_cleanup(m + [None] * (16 - k), asc=True)
    return [w[k - 1 - i] for i in range(k)]


def _merge_bot(a, b):
    k = N_BOTTOM
    m = [jnp.minimum(a[i], b[k - 1 - i]) for i in range(k)]
    w = _bitonic_cleanup(m + [None] * (16 - k), asc=False)
    return [w[k - 1 - i] for i in range(k)]


def _extreme_block(x_ref, o_ref, t_scr, b_scr, acc_t, acc_b):
    for g in range(N_GROUPS):
        base = g * GROUP * LANES
        tiles = [x_ref[:, base + t * LANES:base + (t + 1) * LANES]
                 for t in range(GROUP)]
        for i, j in _SORT16:
            a, b = tiles[i], tiles[j]
            tiles[i] = jnp.minimum(a, b)
            tiles[j] = jnp.maximum(a, b)
        sl = slice(g * LANES, (g + 1) * LANES)
        for i in range(N_TOP):
            t_scr[i, :, sl] = tiles[GROUP - 1 - i]
        for i in range(N_BOTTOM):
            b_scr[i, :, sl] = tiles[i]

    tops = [t_scr[i] for i in range(N_TOP)]
    bots = [b_scr[i] for i in range(N_BOTTOM)]
    width = LIST_W
    while width > LANES:
        half = width // 2
        tops = _merge_top([t[:, :half] for t in tops],
                          [t[:, half:width] for t in tops])
        bots = _merge_bot([b[:, :half] for b in bots],
                          [b[:, half:width] for b in bots])
        width = half

    blk = pl.program_id(0)
    row0 = blk * ROWS_PER_BLOCK
    for i in range(N_TOP):
        acc_t[i, pl.ds(row0, ROWS_PER_BLOCK), :] = tops[i]
    for i in range(N_BOTTOM):
        acc_b[i, pl.ds(row0, ROWS_PER_BLOCK), :] = bots[i]

    @pl.when(blk == pl.num_programs(0) - 1)
    def _tail():
        tops = [acc_t[i] for i in range(N_TOP)]
        bots = [acc_b[i] for i in range(N_BOTTOM)]
        width = LANES
        while width > 1:
            half = width // 2
            tops = _merge_top([t[:, :half] for t in tops],
                              [t[:, half:width] for t in tops])
            bots = _merge_bot([b[:, :half] for b in bots],
                              [b[:, half:width] for b in bots])
            width = half
        o_ref[...] = jnp.concatenate(tops + bots, axis=1)


@jax.jit
def kernel(x):
    n_rows = x.shape[0]
    grid = (n_rows // ROWS_PER_BLOCK,)
    return pl.pallas_call(
        _extreme_block,
        grid=grid,
        in_specs=[pl.BlockSpec((ROWS_PER_BLOCK, N_COLS), lambda i: (i, 0))],
        out_specs=pl.BlockSpec((n_rows, N_TOP + N_BOTTOM), lambda i: (0, 0)),
        out_shape=jax.ShapeDtypeStruct((n_rows, N_TOP + N_BOTTOM), x.dtype),
        scratch_shapes=[
            pltpu.VMEM((N_TOP, ROWS_PER_BLOCK, LIST_W), jnp.float32),
            pltpu.VMEM((N_BOTTOM, ROWS_PER_BLOCK, LIST_W), jnp.float32),
            pltpu.VMEM((N_TOP, 128, LANES), jnp.float32),
            pltpu.VMEM((N_BOTTOM, 128, LANES), jnp.float32),
        ],
    )(x)


# speedup vs baseline: 1.4930x; 1.0582x over previous
import jax
import jax.numpy as jnp
from jax.experimental import pallas as pl
from jax.experimental.pallas import tpu as pltpu

N_TOP = 10
N_BOTTOM = 10
ROWS_PER_BLOCK = 32
N_COLS = 32768
GROUP = 16
LANES = 128
N_GROUPS = N_COLS // (GROUP * LANES)
LIST_W = N_GROUPS * LANES


def _oems_pairs(n):
    pairs = []

    def merge(lo, m, r):
        step = r * 2
        if step < m:
            merge(lo, m, step)
            merge(lo + r, m, step)
            for i in range(lo + r, lo + m - r, step):
                pairs.append((i, i + r))
        else:
            pairs.append((lo, lo + r))

    def sort(lo, m):
        if m > 1:
            h = m // 2
            sort(lo, h)
            sort(lo + h, h)
            merge(lo, m, 1)

    sort(0, n)
    return pairs


_SORT16 = _oems_pairs(GROUP)


def _bitonic_cleanup(w, asc):
    for d in (8, 4, 2, 1):
        for i in range(16):
            if (i & d) == 0 and i + d < 16:
                a, b = w[i], w[i + d]
                if b is None:
                    continue
                if a is None:
                    w[i], w[i + d] = b, None
                    continue
                if asc:
                    w[i], w[i + d] = jnp.minimum(a, b), jnp.maximum(a, b)
                else:
                    w[i], w[i + d] = jnp.maximum(a, b), jnp.minimum(a, b)
    return w


def _merge_top(a, b):
    k = N_TOP
    m = [jnp.maximum(a[i], b[k - 1 - i]) for i in range(k)]
    w = _bitonic_cleanup(m + [None] * (16 - k), asc=True)
    return [w[k - 1 - i] for i in range(k)]


def _merge_bot(a, b):
    k = N_BOTTOM
    m = [jnp.minimum(a[i], b[k - 1 - i]) for i in range(k)]
    w = _bitonic_cleanup(m + [None] * (16 - k), asc=False)
    return [w[k - 1 - i] for i in range(k)]


def _extreme_block(x_ref, o_ref, t_scr, b_scr, acc_t, acc_b):
    for g in range(N_GROUPS):
        base = g * GROUP * LANES
        tiles = [x_ref[:, base + t * LANES:base + (t + 1) * LANES]
                 for t in range(GROUP)]
        for i, j in _SORT16:
            a, b = tiles[i], tiles[j]
            tiles[i] = jnp.minimum(a, b)
            tiles[j] = jnp.maximum(a, b)
        sl = slice(g * LANES, (g + 1) * LANES)
        for i in range(N_TOP):
            t_scr[i, :, sl] = tiles[GROUP - 1 - i]
        for i in range(N_BOTTOM):
            b_scr[i, :, sl] = tiles[i]

    tops = [t_scr[i] for i in range(N_TOP)]
    bots = [b_scr[i] for i in range(N_BOTTOM)]
    width = LIST_W
    while width > LANES:
        half = width // 2
        tops = _merge_top([t[:, :half] for t in tops],
                          [t[:, half:width] for t in tops])
        bots = _merge_bot([b[:, :half] for b in bots],
                          [b[:, half:width] for b in bots])
        width = half

    blk = pl.program_id(0)
    row0 = blk * ROWS_PER_BLOCK
    for i in range(N_TOP):
        acc_t[i, pl.ds(row0, ROWS_PER_BLOCK), :] = tops[i]
    for i in range(N_BOTTOM):
        acc_b[i, pl.ds(row0, ROWS_PER_BLOCK), :] = bots[i]

    @pl.when(blk == pl.num_programs(0) - 1)
    def _tail():
        tops = [acc_t[i] for i in range(N_TOP)]
        bots = [acc_b[i] for i in range(N_BOTTOM)]
        width = LANES
        while width > 1:
            half = width // 2
            tops = _merge_top([t[:, :half] for t in tops],
                              [t[:, half:width] for t in tops])
            bots = _merge_bot([b[:, :half] for b in bots],
                              [b[:, half:width] for b in bots])
            width = half
        o_ref[...] = jnp.concatenate(tops + bots, axis=1)


@jax.jit
def kernel(x):
    n_rows = x.shape[0]
    grid = (n_rows // ROWS_PER_BLOCK,)
    return pl.pallas_call(
        _extreme_block,
        grid=grid,
        in_specs=[pl.BlockSpec((ROWS_PER_BLOCK, N_COLS), lambda i: (i, 0))],
        out_specs=pl.BlockSpec((n_rows, N_TOP + N_BOTTOM), lambda i: (0, 0)),
        out_shape=jax.ShapeDtypeStruct((n_rows, N_TOP + N_BOTTOM), x.dtype),
        scratch_shapes=[
            pltpu.VMEM((N_TOP, ROWS_PER_BLOCK, LIST_W), jnp.float32),
            pltpu.VMEM((N_BOTTOM, ROWS_PER_BLOCK, LIST_W), jnp.float32),
            pltpu.VMEM((N_TOP, 128, LANES), jnp.float32),
            pltpu.VMEM((N_BOTTOM, 128, LANES), jnp.float32),
        ],
    )(x)


# speedup vs baseline: 1.5096x; 1.0112x over previous
import jax
import jax.numpy as jnp
from jax.experimental import pallas as pl
from jax.experimental.pallas import tpu as pltpu

N_TOP = 10
N_BOTTOM = 10
N_ROWS = 128
ROWS_PER_BLOCK = 32
PANEL = 8
N_COLS = 32768
GROUP = 16
LANES = 128
N_GROUPS = N_COLS // (GROUP * LANES)
LIST_W = N_GROUPS * LANES


def _oems_pairs(n):
    pairs = []

    def merge(lo, m, r):
        step = r * 2
        if step < m:
            merge(lo, m, step)
            merge(lo + r, m, step)
            for i in range(lo + r, lo + m - r, step):
                pairs.append((i, i + r))
        else:
            pairs.append((lo, lo + r))

    def sort(lo, m):
        if m > 1:
            h = m // 2
            sort(lo, h)
            sort(lo + h, h)
            merge(lo, m, 1)

    sort(0, n)
    return pairs


_SORT16 = _oems_pairs(GROUP)


def _bitonic_cleanup(w, asc):
    for d in (8, 4, 2, 1):
        for i in range(16):
            if (i & d) == 0 and i + d < 16:
                a, b = w[i], w[i + d]
                if b is None:
                    continue
                if a is None:
                    w[i], w[i + d] = b, None
                    continue
                if asc:
                    w[i], w[i + d] = jnp.minimum(a, b), jnp.maximum(a, b)
                else:
                    w[i], w[i + d] = jnp.maximum(a, b), jnp.minimum(a, b)
    return w


def _merge_top(a, b):
    k = N_TOP
    m = [jnp.maximum(a[i], b[k - 1 - i]) for i in range(k)]
    w = _bitonic_cleanup(m + [None] * (16 - k), asc=True)
    return [w[k - 1 - i] for i in range(k)]


def _merge_bot(a, b):
    k = N_BOTTOM
    m = [jnp.minimum(a[i], b[k - 1 - i]) for i in range(k)]
    w = _bitonic_cleanup(m + [None] * (16 - k), asc=False)
    return [w[k - 1 - i] for i in range(k)]


def _extreme_block(x_ref, o_ref, t_scr, b_scr, acc_t, acc_b):
    blk = pl.program_id(0)
    for p in range(ROWS_PER_BLOCK // PANEL):
        rows = slice(p * PANEL, (p + 1) * PANEL)
        for g in range(N_GROUPS):
            base = g * GROUP * LANES
            tiles = [x_ref[rows, base + t * LANES:base + (t + 1) * LANES]
                     for t in range(GROUP)]
            for i, j in _SORT16:
                a, b = tiles[i], tiles[j]
                tiles[i] = jnp.minimum(a, b)
                tiles[j] = jnp.maximum(a, b)
            sl = slice(g * LANES, (g + 1) * LANES)
            for i in range(N_TOP):
                t_scr[i, rows, sl] = tiles[GROUP - 1 - i]
            for i in range(N_BOTTOM):
                b_scr[i, rows, sl] = tiles[i]

        tops = [t_scr[i, rows, :] for i in range(N_TOP)]
        bots = [b_scr[i, rows, :] for i in range(N_BOTTOM)]
        width = LIST_W
        while width > LANES:
            half = width // 2
            tops = _merge_top([t[:, :half] for t in tops],
                              [t[:, half:width] for t in tops])
            bots = _merge_bot([b[:, :half] for b in bots],
                              [b[:, half:width] for b in bots])
            width = half

        row0 = blk * ROWS_PER_BLOCK + p * PANEL
        for i in range(N_TOP):
            acc_t[i, pl.ds(row0, PANEL), :] = tops[i]
        for i in range(N_BOTTOM):
            acc_b[i, pl.ds(row0, PANEL), :] = bots[i]

    @pl.when(blk == pl.num_programs(0) - 1)
    def _tail():
        tops = [acc_t[i] for i in range(N_TOP)]
        bots = [acc_b[i] for i in range(N_BOTTOM)]
        width = LANES
        while width > 1:
            half = width // 2
            tops = _merge_top([t[:, :half] for t in tops],
                              [t[:, half:width] for t in tops])
            bots = _merge_bot([b[:, :half] for b in bots],
                              [b[:, half:width] for b in bots])
            width = half
        o_ref[...] = jnp.concatenate(tops + bots, axis=1)


@jax.jit
def kernel(x):
    n_rows = x.shape[0]
    grid = (n_rows // ROWS_PER_BLOCK,)
    return pl.pallas_call(
        _extreme_block,
        grid=grid,
        in_specs=[pl.BlockSpec((ROWS_PER_BLOCK, N_COLS), lambda i: (i, 0))],
        out_specs=pl.BlockSpec((n_rows, N_TOP + N_BOTTOM), lambda i: (0, 0)),
        out_shape=jax.ShapeDtypeStruct((n_rows, N_TOP + N_BOTTOM), x.dtype),
        scratch_shapes=[
            pltpu.VMEM((N_TOP, ROWS_PER_BLOCK, LIST_W), jnp.float32),
            pltpu.VMEM((N_BOTTOM, ROWS_PER_BLOCK, LIST_W), jnp.float32),
            pltpu.VMEM((N_TOP, N_ROWS, LANES), jnp.float32),
            pltpu.VMEM((N_BOTTOM, N_ROWS, LANES), jnp.float32),
        ],
    )(x)


# speedup vs baseline: 1.5587x; 1.0325x over previous
import jax
import jax.numpy as jnp
from jax.experimental import pallas as pl
from jax.experimental.pallas import tpu as pltpu

N_TOP = 10
N_BOTTOM = 10
N_ROWS = 128
ROWS_PER_BLOCK = 32
PANEL = 8
N_COLS = 32768
GROUP = 16
LANES = 128
N_GROUPS = N_COLS // (GROUP * LANES)
LIST_W = N_GROUPS * LANES


def _oems_pairs(n):
    pairs = []

    def merge(lo, m, r):
        step = r * 2
        if step < m:
            merge(lo, m, step)
            merge(lo + r, m, step)
            for i in range(lo + r, lo + m - r, step):
                pairs.append((i, i + r))
        else:
            pairs.append((lo, lo + r))

    def sort(lo, m):
        if m > 1:
            h = m // 2
            sort(lo, h)
            sort(lo + h, h)
            merge(lo, m, 1)

    sort(0, n)
    return pairs


_SORT16 = _oems_pairs(GROUP)


def _bitonic_cleanup(w, asc):
    for d in (8, 4, 2, 1):
        for i in range(16):
            if (i & d) == 0 and i + d < 16:
                a, b = w[i], w[i + d]
                if b is None:
                    continue
                if a is None:
                    w[i], w[i + d] = b, None
                    continue
                if asc:
                    w[i], w[i + d] = jnp.minimum(a, b), jnp.maximum(a, b)
                else:
                    w[i], w[i + d] = jnp.maximum(a, b), jnp.minimum(a, b)
    return w


def _merge_top(a, b):
    k = N_TOP
    m = [jnp.maximum(a[i], b[k - 1 - i]) for i in range(k)]
    w = _bitonic_cleanup(m + [None] * (16 - k), asc=True)
    return [w[k - 1 - i] for i in range(k)]


def _merge_bot(a, b):
    k = N_BOTTOM
    m = [jnp.minimum(a[i], b[k - 1 - i]) for i in range(k)]
    w = _bitonic_cleanup(m + [None] * (16 - k), asc=False)
    return [w[k - 1 - i] for i in range(k)]


def _extreme_block(x_ref, o_ref, acc_t, acc_b):
    blk = pl.program_id(0)
    for p in range(ROWS_PER_BLOCK // PANEL):
        rows = slice(p * PANEL, (p + 1) * PANEL)
        tops = None
        bots = None
        for g in range(N_GROUPS):
            base = g * GROUP * LANES
            tiles = [x_ref[rows, base + t * LANES:base + (t + 1) * LANES]
                     for t in range(GROUP)]
            for i, j in _SORT16:
                a, b = tiles[i], tiles[j]
                tiles[i] = jnp.minimum(a, b)
                tiles[j] = jnp.maximum(a, b)
            g_top = [tiles[GROUP - 1 - i] for i in range(N_TOP)]
            g_bot = tiles[:N_BOTTOM]
            if tops is None:
                tops, bots = g_top, g_bot
            else:
                tops = _merge_top(tops, g_top)
                bots = _merge_bot(bots, g_bot)

        row0 = blk * ROWS_PER_BLOCK + p * PANEL
        for i in range(N_TOP):
            acc_t[i, pl.ds(row0, PANEL), :] = tops[i]
        for i in range(N_BOTTOM):
            acc_b[i, pl.ds(row0, PANEL), :] = bots[i]

    @pl.when(blk == pl.num_programs(0) - 1)
    def _tail():
        tops = [acc_t[i] for i in range(N_TOP)]
        bots = [acc_b[i] for i in range(N_BOTTOM)]
        width = LANES
        while width > 1:
            half = width // 2
            tops = _merge_top([t[:, :half] for t in tops],
                              [t[:, half:width] for t in tops])
            bots = _merge_bot([b[:, :half] for b in bots],
                              [b[:, half:width] for b in bots])
            width = half
        o_ref[...] = jnp.concatenate(tops + bots, axis=1)


@jax.jit
def kernel(x):
    n_rows = x.shape[0]
    grid = (n_rows // ROWS_PER_BLOCK,)
    return pl.pallas_call(
        _extreme_block,
        grid=grid,
        in_specs=[pl.BlockSpec((ROWS_PER_BLOCK, N_COLS), lambda i: (i, 0))],
        out_specs=pl.BlockSpec((n_rows, N_TOP + N_BOTTOM), lambda i: (0, 0)),
        out_shape=jax.ShapeDtypeStruct((n_rows, N_TOP + N_BOTTOM), x.dtype),
        scratch_shapes=[
            pltpu.VMEM((N_TOP, N_ROWS, LANES), jnp.float32),
            pltpu.VMEM((N_BOTTOM, N_ROWS, LANES), jnp.float32),
        ],
    )(x)


# speedup vs baseline: 1.9170x; 1.2298x over previous
import jax
import jax.numpy as jnp
from jax.experimental import pallas as pl
from jax.experimental.pallas import tpu as pltpu

N_TOP = 10
N_BOTTOM = 10
N_ROWS = 128
ROWS_PER_BLOCK = 32
PANEL = 8
N_COLS = 32768
GROUP = 16
LANES = 128
N_GROUPS = N_COLS // (GROUP * LANES)
LIST_W = N_GROUPS * LANES


def _oems_pairs(n):
    pairs = []

    def merge(lo, m, r):
        step = r * 2
        if step < m:
            merge(lo, m, step)
            merge(lo + r, m, step)
            for i in range(lo + r, lo + m - r, step):
                pairs.append((i, i + r))
        else:
            pairs.append((lo, lo + r))

    def sort(lo, m):
        if m > 1:
            h = m // 2
            sort(lo, h)
            sort(lo + h, h)
            merge(lo, m, 1)

    sort(0, n)
    return pairs


_SORT16 = _oems_pairs(GROUP)


def _bitonic_cleanup(w, asc):
    for d in (8, 4, 2, 1):
        for i in range(16):
            if (i & d) == 0 and i + d < 16:
                a, b = w[i], w[i + d]
                if b is None:
                    continue
                if a is None:
                    w[i], w[i + d] = b, None
                    continue
                if asc:
                    w[i], w[i + d] = jnp.minimum(a, b), jnp.maximum(a, b)
                else:
                    w[i], w[i + d] = jnp.maximum(a, b), jnp.minimum(a, b)
    return w


def _merge_top(a, b):
    k = N_TOP
    m = [jnp.maximum(a[i], b[k - 1 - i]) for i in range(k)]
    w = _bitonic_cleanup(m + [None] * (16 - k), asc=True)
    return [w[k - 1 - i] for i in range(k)]


def _merge_bot(a, b):
    k = N_BOTTOM
    m = [jnp.minimum(a[i], b[k - 1 - i]) for i in range(k)]
    w = _bitonic_cleanup(m + [None] * (16 - k), asc=False)
    return [w[k - 1 - i] for i in range(k)]


def _extreme_block(x_ref, o_ref, acc_t, acc_b):
    blk = pl.program_id(0)
    for p in range(ROWS_PER_BLOCK // PANEL):
        rows = slice(p * PANEL, (p + 1) * PANEL)
        tops = None
        bots = None
        for g in range(N_GROUPS):
            base = g * GROUP * LANES
            tiles = [x_ref[rows, base + t * LANES:base + (t + 1) * LANES]
                     for t in range(GROUP)]
            for i, j in _SORT16:
                a, b = tiles[i], tiles[j]
                tiles[i] = jnp.minimum(a, b)
                tiles[j] = jnp.maximum(a, b)
            g_top = [tiles[GROUP - 1 - i] for i in range(N_TOP)]
            g_bot = tiles[:N_BOTTOM]
            if tops is None:
                tops, bots = g_top, g_bot
            else:
                tops = _merge_top(tops, g_top)
                bots = _merge_bot(bots, g_bot)

        row0 = blk * ROWS_PER_BLOCK + p * PANEL
        for i in range(N_TOP):
            acc_t[i, pl.ds(row0, PANEL), :] = tops[i]
        for i in range(N_BOTTOM):
            acc_b[i, pl.ds(row0, PANEL), :] = bots[i]

    @pl.when(blk == pl.num_programs(0) - 1)
    def _tail():
        tops = [jnp.transpose(acc_t[i]) for i in range(N_TOP)]
        bots = [jnp.transpose(acc_b[i]) for i in range(N_BOTTOM)]
        width = LANES
        while width > 1:
            half = width // 2
            tops = _merge_top([t[:half, :] for t in tops],
                              [t[half:width, :] for t in tops])
            bots = _merge_bot([b[:half, :] for b in bots],
                              [b[half:width, :] for b in bots])
            width = half
        out = jnp.concatenate(tops + bots, axis=0)
        o_ref[...] = jnp.transpose(out)


@jax.jit
def kernel(x):
    n_rows = x.shape[0]
    grid = (n_rows // ROWS_PER_BLOCK,)
    return pl.pallas_call(
        _extreme_block,
        grid=grid,
        in_specs=[pl.BlockSpec((ROWS_PER_BLOCK, N_COLS), lambda i: (i, 0))],
        out_specs=pl.BlockSpec((n_rows, N_TOP + N_BOTTOM), lambda i: (0, 0)),
        out_shape=jax.ShapeDtypeStruct((n_rows, N_TOP + N_BOTTOM), x.dtype),
        scratch_shapes=[
            pltpu.VMEM((N_TOP, N_ROWS, LANES), jnp.float32),
            pltpu.VMEM((N_BOTTOM, N_ROWS, LANES), jnp.float32),
        ],
    )(x)
